# Initial kernel scaffold; baseline (speedup 1.0000x reference)
import jax
import jax.numpy as jnp
from jax.experimental import pallas as pl


def kernel(word, word_embeddings, grammar_preterminates, W1, b1):
    raise NotImplementedError("write your pallas kernel here")


# speedup vs baseline: 5.7949x; 5.7949x over previous
import functools

import jax
import jax.numpy as jnp
from jax import lax
from jax.experimental import pallas as pl
from jax.experimental.pallas import tpu as pltpu
from jax.experimental.pallas import tpu_sc as plsc

N_TOK = 32768
D_EMB = 512
S_DIM = 256

NC = 2
NS = 16
NW = NC * NS
B_PER_W = N_TOK // NW
CHUNK = 64
NCHUNK = B_PER_W // CHUNK


def _sc_gather(word_chunks, table):
    mesh = plsc.VectorSubcoreMesh(core_axis_name="c", subcore_axis_name="s")

    @functools.partial(
        pl.kernel,
        mesh=mesh,
        out_type=jax.ShapeDtypeStruct((N_TOK, D_EMB), jnp.float32),
        scratch_types=[
            pltpu.VMEM((NCHUNK, CHUNK), jnp.int32),
            pltpu.VMEM((2, CHUNK, D_EMB), jnp.float32),
            pltpu.SemaphoreType.DMA,
            pltpu.SemaphoreType.DMA,
        ],
    )
    def k(idx_hbm, table_hbm, out_hbm, idx_v, bufs, sem0, sem1):
        wid = lax.axis_index("s") * NC + lax.axis_index("c")
        base = wid * B_PER_W
        pltpu.sync_copy(idx_hbm.at[wid], idx_v)
        sems = [sem0, sem1]
        cps = [None, None]
        cps[0] = pltpu.async_copy(
            table_hbm.at[idx_v.at[0]], bufs.at[0], sems[0])
        for c in range(NCHUNK):
            if c + 1 < NCHUNK:
                cps[(c + 1) % 2] = pltpu.async_copy(
                    table_hbm.at[idx_v.at[c + 1]],
                    bufs.at[(c + 1) % 2],
                    sems[(c + 1) % 2])
            cps[c % 2].wait()
            pltpu.sync_copy(bufs.at[c % 2],
                            out_hbm.at[pl.ds(base + c * CHUNK, CHUNK)])

    return k(word_chunks, table)


def _tc_mlp(x, W1, b1):
    BM = 1024

    def body(x_ref, w_ref, b_ref, o_ref):
        acc = jnp.dot(x_ref[...], w_ref[...],
                      preferred_element_type=jnp.float32)
        o_ref[...] = jnp.tanh(acc + b_ref[...])

    return pl.pallas_call(
        body,
        grid=(N_TOK // BM,),
        in_specs=[
            pl.BlockSpec((BM, D_EMB), lambda i: (i, 0)),
            pl.BlockSpec((D_EMB, S_DIM), lambda i: (0, 0)),
            pl.BlockSpec((1, S_DIM), lambda i: (0, 0)),
        ],
        out_specs=pl.BlockSpec((BM, S_DIM), lambda i: (i, 0)),
        out_shape=jax.ShapeDtypeStruct((N_TOK, S_DIM), jnp.float32),
    )(x, W1, b1.reshape(1, S_DIM))


def kernel(word, word_embeddings, grammar_preterminates, W1, b1):
    del grammar_preterminates
    word_chunks = word.astype(jnp.int32).reshape(NW, NCHUNK, CHUNK)
    gathered = _sc_gather(word_chunks, word_embeddings)
    return _tc_mlp(gathered, W1, b1)
